# baseline (speedup 1.0000x reference)
import jax
import jax.numpy as jnp
import numpy as np
from jax.experimental import pallas as pl
from jax.experimental.pallas import tpu as pltpu

_PERCENTAGE = 0.5
_SEED = 0
_B = 16

def _subset_rows():
    try:
        cpu = jax.devices("cpu")[0]
        with jax.default_device(cpu):
            perm = np.asarray(jax.random.permutation(jax.random.key(_SEED), _B))
        sel = sorted(int(v) for v in perm[: int(_B * _PERCENTAGE)])
    except Exception:
        sel = [0, 1, 4, 5, 6, 8, 12, 13]
    unsel = sorted(set(range(_B)) - set(sel))
    return tuple(sel), tuple(unsel)

_SEL, _UNSEL = _subset_rows()

_R = 96
_C = 224 * 224
_RB = 16
_NCHUNK = _R // _RB
_NSTEP = len(_SEL) * _NCHUNK


def _body(sel_ref, x_ref, x_any, o_any, scratch, copy_sem, store_sem):
    i = pl.program_id(0)
    r = pl.program_id(1)
    k = i * _NCHUNK + r
    slot = jax.lax.rem(k, 2)

    @pl.when(k == 0)
    def _():
        for u, row in enumerate(_UNSEL):
            pltpu.make_async_copy(
                x_any.at[row], o_any.at[row], copy_sem.at[u]
            ).start()

    @pl.when(k >= 2)
    def _():
        pltpu.make_async_copy(
            scratch.at[slot], o_any.at[0, pl.ds(0, _RB), :], store_sem.at[slot]
        ).wait()

    scratch[slot] = jnp.maximum(x_ref[0], 0.0)

    row = sel_ref[i]
    pltpu.make_async_copy(
        scratch.at[slot], o_any.at[row, pl.ds(r * _RB, _RB), :],
        store_sem.at[slot],
    ).start()

    @pl.when(k == _NSTEP - 1)
    def _():
        pltpu.make_async_copy(
            scratch.at[slot], o_any.at[0, pl.ds(0, _RB), :], store_sem.at[slot]
        ).wait()
        other = 1 - slot
        pltpu.make_async_copy(
            scratch.at[other], o_any.at[0, pl.ds(0, _RB), :], store_sem.at[other]
        ).wait()
        for u, row_u in enumerate(_UNSEL):
            pltpu.make_async_copy(
                x_any.at[row_u], o_any.at[row_u], copy_sem.at[u]
            ).wait()


def kernel(x):
    xv = x.reshape(_B, _R, _C)
    sel_arr = jnp.asarray(_SEL, dtype=jnp.int32)
    out = pl.pallas_call(
        _body,
        grid_spec=pltpu.PrefetchScalarGridSpec(
            num_scalar_prefetch=1,
            grid=(len(_SEL), _NCHUNK),
            in_specs=[
                pl.BlockSpec((1, _RB, _C), lambda i, r, sel: (sel[i], r, 0)),
                pl.BlockSpec(memory_space=pl.ANY),
            ],
            out_specs=pl.BlockSpec(memory_space=pl.ANY),
            scratch_shapes=[
                pltpu.VMEM((2, _RB, _C), jnp.float32),
                pltpu.SemaphoreType.DMA((len(_UNSEL),)),
                pltpu.SemaphoreType.DMA((2,)),
            ],
        ),
        out_shape=jax.ShapeDtypeStruct((_B, _R, _C), jnp.float32),
    )(sel_arr, xv, xv)
    return out.reshape(x.shape)
